# baseline (speedup 1.0000x reference)
import jax
import jax.numpy as jnp
from jax import lax
from jax.experimental import pallas as pl
from jax.experimental.pallas import tpu as pltpu
from jax.experimental.pallas import tpu_sc as plsc

L = 16
NW = 32
D = 32
B = 16384
BPW = B // NW
CH = 128
NCH = BPW // CH


def _body(user_hbm, item_hbm, uf_hbm, if_hbm, ub_hbm, ib_hbm, gb_hbm, out_hbm,
          uidx_v, iidx_v, urows_v, irows_v, ub_v, ib_v, gb_v, out_v, sem):
    c = lax.axis_index("c")
    s = lax.axis_index("s")
    wid = s * 2 + c
    rbase = wid * NCH

    pltpu.sync_copy(user_hbm.at[pl.ds(rbase, NCH)], uidx_v)
    pltpu.sync_copy(item_hbm.at[pl.ds(rbase, NCH)], iidx_v)
    pltpu.sync_copy(gb_hbm, gb_v)

    copies = []
    for k in range(NCH):
        dst = pl.ds(k * CH, CH)
        copies.append(pltpu.async_copy(uf_hbm.at[uidx_v.at[k]], urows_v.at[dst], sem))
        copies.append(pltpu.async_copy(if_hbm.at[iidx_v.at[k]], irows_v.at[dst], sem))
        copies.append(pltpu.async_copy(ub_hbm.at[uidx_v.at[k]], ub_v.at[dst], sem))
        copies.append(pltpu.async_copy(ib_hbm.at[iidx_v.at[k]], ib_v.at[dst], sem))
    for cp in copies:
        cp.wait()

    gb_vec = gb_v[...]
    zeros = jnp.zeros((L,), jnp.int32)

    def group(g, carry):
        rows = lax.iota(jnp.int32, L) + g * L
        acc = gb_vec
        acc = acc + ub_v[pl.ds(g * L, L)]
        acc = acc + ib_v[pl.ds(g * L, L)]
        for j in range(D):
            cj = jnp.full((L,), j, jnp.int32)
            acc = acc + (plsc.load_gather(urows_v, [rows, cj])
                         * plsc.load_gather(irows_v, [rows, cj]))
        out_v[pl.ds(g * L, L)] = acc
        return carry

    lax.fori_loop(0, BPW // L, group, 0)
    pltpu.sync_copy(out_v, out_hbm.at[pl.ds(wid * BPW, BPW)])


def kernel(user, item, user_factors, item_factors, user_bias, item_bias, global_bias):
    user2 = user.reshape(B // CH, CH).astype(jnp.int32)
    item2 = item.reshape(B // CH, CH).astype(jnp.int32)
    gb16 = jnp.broadcast_to(global_bias.astype(jnp.float32), (L,))
    ub_flat = user_bias.reshape(-1)
    ib_flat = item_bias.reshape(-1)
    mesh = plsc.VectorSubcoreMesh(core_axis_name="c", subcore_axis_name="s")
    k = pl.kernel(
        _body,
        mesh=mesh,
        out_type=jax.ShapeDtypeStruct((B,), jnp.float32),
        compiler_params=pltpu.CompilerParams(
            needs_layout_passes=False, use_tc_tiling_on_sc=False),
        scratch_types=[
            pltpu.VMEM((NCH, CH), jnp.int32),
            pltpu.VMEM((NCH, CH), jnp.int32),
            pltpu.VMEM((BPW, D), jnp.float32),
            pltpu.VMEM((BPW, D), jnp.float32),
            pltpu.VMEM((BPW,), jnp.float32),
            pltpu.VMEM((BPW,), jnp.float32),
            pltpu.VMEM((L,), jnp.float32),
            pltpu.VMEM((BPW,), jnp.float32),
            pltpu.SemaphoreType.DMA,
        ],
    )
    return k(user2, item2, user_factors, item_factors, ub_flat, ib_flat,
             gb16)


# speedup vs baseline: 2.5767x; 2.5767x over previous
import jax
import jax.numpy as jnp
from jax import lax
from jax.experimental import pallas as pl
from jax.experimental.pallas import tpu as pltpu
from jax.experimental.pallas import tpu_sc as plsc

L = 16
NW = 32
D = 32
B = 16384
BPW = B // NW
CH = 128
NCH = BPW // CH
Q = 4


def _body(user_hbm, item_hbm, uft_hbm, ift_hbm, ub_hbm, ib_hbm, gb_hbm, out_hbm,
          uidx_v, iidx_v, upan, ipan, ubuf, ibuf, ubias_v, ibias_v,
          gb_v, out_v, sem0, sem1, bsem):
    c = lax.axis_index("c")
    s = lax.axis_index("s")
    wid = s * 2 + c
    base = wid * BPW

    pltpu.sync_copy(user_hbm.at[pl.ds(base, BPW)], uidx_v)
    pltpu.sync_copy(item_hbm.at[pl.ds(base, BPW)], iidx_v)
    pltpu.sync_copy(gb_hbm, gb_v)

    bias_cps = []
    for k in range(NCH):
        dst = pl.ds(k * CH, CH)
        bias_cps.append(
            pltpu.async_copy(ub_hbm.at[uidx_v.at[dst]], ubias_v.at[dst], bsem))
        bias_cps.append(
            pltpu.async_copy(ib_hbm.at[iidx_v.at[dst]], ibias_v.at[dst], bsem))

    d_all = pl.ds(0, D)
    jlo = lax.iota(jnp.int32, L)
    jhi = jlo + L
    sems = (sem0, sem1)

    def issue(e_vec, lane, slot, k):
        u = e_vec[0][lane]
        t = e_vec[1][lane]
        qu = pl.multiple_of((u >> 7) << 7, 128)
        qt = pl.multiple_of((t >> 7) << 7, 128)
        pltpu.async_copy(uft_hbm.at[d_all, pl.ds(qu, CH)],
                         upan.at[slot * Q + k], sems[slot])
        pltpu.async_copy(ift_hbm.at[d_all, pl.ds(qt, CH)],
                         ipan.at[slot * Q + k], sems[slot])

    def drain(slot):
        for k in range(Q):
            pltpu.make_async_copy(uft_hbm.at[d_all, pl.ds(0, CH)],
                                  upan.at[slot * Q + k], sems[slot]).wait()
            pltpu.make_async_copy(uft_hbm.at[d_all, pl.ds(0, CH)],
                                  ipan.at[slot * Q + k], sems[slot]).wait()

    def extract(e_vec, lane, slot, k, r):
        mu = jnp.full((L,), e_vec[0][lane] & 127, jnp.int32)
        mt = jnp.full((L,), e_vec[1][lane] & 127, jnp.int32)
        up = upan.at[slot * Q + k]
        ip = ipan.at[slot * Q + k]
        ubuf[pl.ds(r, L)] = plsc.load_gather(up, [jlo, mu])
        ubuf[pl.ds(r + L, L)] = plsc.load_gather(up, [jhi, mu])
        ibuf[pl.ds(r, L)] = plsc.load_gather(ip, [jlo, mt])
        ibuf[pl.ds(r + L, L)] = plsc.load_gather(ip, [jhi, mt])

    def pipe(p, carry):
        pvu, pvi = carry
        vu = uidx_v[pl.ds(p * L, L)]
        vi = iidx_v[pl.ds(p * L, L)]
        cur = (vu, vi)
        prev = (pvu, pvi)
        for qq in range(4):
            slot = qq % 2
            for k in range(Q):
                issue(cur, qq * Q + k, slot, k)
            pslot = (qq + 1) % 2
            if qq == 0:
                @pl.when(p > 0)
                def _():
                    drain(pslot)
                    for k in range(Q):
                        r = ((p - 1) * L + 3 * Q + k) * D
                        extract(prev, 3 * Q + k, pslot, k, r)
            else:
                drain(pslot)
                for k in range(Q):
                    r = (p * L + (qq - 1) * Q + k) * D
                    extract(cur, (qq - 1) * Q + k, pslot, k, r)
        return cur

    zero16 = jnp.zeros((L,), jnp.int32)
    lvu, lvi = lax.fori_loop(0, BPW // L, pipe, (zero16, zero16))
    drain(1)
    last = (lvu, lvi)
    for k in range(Q):
        r = ((BPW // L - 1) * L + 3 * Q + k) * D
        extract(last, 3 * Q + k, 1, k, r)
    for cp in bias_cps:
        cp.wait()

    gb_vec = gb_v[...]
    lane32 = lax.iota(jnp.int32, L) * D

    def group(g, carry):
        l0 = g * L
        acc = gb_vec + ubias_v[pl.ds(l0, L)] + ibias_v[pl.ds(l0, L)]
        for j in range(D):
            idx = lane32 + (g * (L * D) + j)
            acc = acc + plsc.load_gather(ubuf, [idx]) * plsc.load_gather(ibuf, [idx])
        out_v[pl.ds(l0, L)] = acc
        return carry

    lax.fori_loop(0, BPW // L, group, 0)
    pltpu.sync_copy(out_v, out_hbm.at[pl.ds(base, BPW)])


def kernel(user, item, user_factors, item_factors, user_bias, item_bias, global_bias):
    uft = user_factors.T
    ift = item_factors.T
    ub_flat = user_bias.reshape(-1)
    ib_flat = item_bias.reshape(-1)
    gb16 = jnp.broadcast_to(global_bias.astype(jnp.float32), (L,))
    mesh = plsc.VectorSubcoreMesh(core_axis_name="c", subcore_axis_name="s")
    k = pl.kernel(
        _body,
        mesh=mesh,
        out_type=jax.ShapeDtypeStruct((B,), jnp.float32),
        compiler_params=pltpu.CompilerParams(needs_layout_passes=False),
        scratch_types=[
            pltpu.VMEM((BPW,), jnp.int32),
            pltpu.VMEM((BPW,), jnp.int32),
            pltpu.VMEM((2 * Q, D, CH), jnp.float32),
            pltpu.VMEM((2 * Q, D, CH), jnp.float32),
            pltpu.VMEM((BPW * D,), jnp.float32),
            pltpu.VMEM((BPW * D,), jnp.float32),
            pltpu.VMEM((BPW,), jnp.float32),
            pltpu.VMEM((BPW,), jnp.float32),
            pltpu.VMEM((L,), jnp.float32),
            pltpu.VMEM((BPW,), jnp.float32),
            pltpu.SemaphoreType.DMA,
            pltpu.SemaphoreType.DMA,
            pltpu.SemaphoreType.DMA,
        ],
    )
    return k(user.astype(jnp.int32), item.astype(jnp.int32), uft, ift,
             ub_flat, ib_flat, gb16)
